# baseline (speedup 1.0000x reference)
import functools

import jax
import jax.numpy as jnp
from jax import lax
from jax.experimental import pallas as pl
from jax.experimental.pallas import tpu as pltpu
from jax.experimental.pallas import tpu_sc as plsc

_WS = 16
_T = _WS * _WS
_H = 16
_NV = (2 * _WS - 1) ** 2
_NC = 2
_NS = 16
_NW = _NC * _NS
_R = _T // _NW
_L = 16
_G = _T // _L


def _make_mesh():
    return plsc.VectorSubcoreMesh(
        core_axis_name="c", subcore_axis_name="s",
        num_cores=_NC, num_subcores=_NS)


@functools.partial(
    pl.kernel,
    out_type=jax.ShapeDtypeStruct((1, _H, _T, _T), jnp.float32),
    mesh=_make_mesh(),
    scratch_types=[
        pltpu.VMEM((_R, _T), jnp.int32),
        pltpu.VMEM((_NV * _H,), jnp.float32),
        pltpu.VMEM((_H, _R, _T), jnp.float32),
        pltpu.SemaphoreType.DMA,
        pltpu.SemaphoreType.DMA,
    ],
    compiler_params=pltpu.CompilerParams(needs_layout_passes=False),
)
def _rpb_gather(tabt_hbm, idx_hbm, out_hbm, idx_v, tab_v, out_v, sem, sem_in):
    wid = lax.axis_index("s") * _NC + lax.axis_index("c")
    row0 = wid * _R
    tab_cp = pltpu.async_copy(tabt_hbm, tab_v, sem_in)
    idx_cp = pltpu.async_copy(idx_hbm.at[pl.ds(row0, _R), :], idx_v, sem_in)
    tab_cp.wait()
    idx_cp.wait()

    copies = []
    for r in range(_R):
        @plsc.parallel_loop(0, _G, unroll=_G)
        def _body(g, r=r):
            off = g * _L
            flat = idx_v[r, pl.ds(off, _L)] * _H
            for h in range(_H):
                out_v[h, r, pl.ds(off, _L)] = plsc.load_gather(
                    tab_v, [flat + h])
        copies.append(pltpu.async_copy(
            out_v.at[:, pl.ds(r, 1), :],
            out_hbm.at[0, :, pl.ds(row0 + r, 1), :], sem))
    for c in copies:
        c.wait()


def kernel(relative_position_bias_table, relative_position_index):
    return _rpb_gather(relative_position_bias_table.reshape(-1),
                       relative_position_index)


# speedup vs baseline: 1.1288x; 1.1288x over previous
import functools

import jax
import jax.numpy as jnp
from jax import lax
from jax.experimental import pallas as pl
from jax.experimental.pallas import tpu as pltpu
from jax.experimental.pallas import tpu_sc as plsc

_WS = 16
_T = _WS * _WS
_H = 16
_NV = (2 * _WS - 1) ** 2
_NC = 2
_NS = 16
_NW = _NC * _NS
_R = _T // _NW
_L = 16
_G = _T // _L


def _make_mesh():
    return plsc.VectorSubcoreMesh(
        core_axis_name="c", subcore_axis_name="s",
        num_cores=_NC, num_subcores=_NS)


@functools.partial(
    pl.kernel,
    out_type=jax.ShapeDtypeStruct((1, _H, _T, _T), jnp.float32),
    mesh=_make_mesh(),
    scratch_types=[
        pltpu.VMEM((_R, _T), jnp.int32),
        pltpu.VMEM((_NV * _H,), jnp.float32),
        pltpu.VMEM((_H, _R, _T), jnp.float32),
        pltpu.SemaphoreType.DMA,
        pltpu.SemaphoreType.DMA,
    ],
    compiler_params=pltpu.CompilerParams(needs_layout_passes=False),
)
def _rpb_gather(tabt_hbm, idx_hbm, out_hbm, idx_v, tab_v, out_v, sem, sem_in):
    wid = lax.axis_index("s") * _NC + lax.axis_index("c")
    row0 = wid * _R
    tab_cp = pltpu.async_copy(tabt_hbm, tab_v, sem_in)
    idx_cp = pltpu.async_copy(idx_hbm.at[pl.ds(row0, _R), :], idx_v, sem_in)
    tab_cp.wait()
    idx_cp.wait()

    @pl.loop(0, _R)
    def _rows(r):
        @plsc.parallel_loop(0, _G, unroll=8)
        def _body(g):
            off = g * _L
            flat = idx_v[r, pl.ds(off, _L)] * _H
            for h in range(_H):
                out_v[h, r, pl.ds(off, _L)] = plsc.load_gather(
                    tab_v, [flat + h])
        pltpu.async_copy(
            out_v.at[:, pl.ds(r, 1), :],
            out_hbm.at[0, :, pl.ds(row0 + r, 1), :], sem)

    for r in range(_R):
        pltpu.make_async_copy(
            out_v.at[:, pl.ds(r, 1), :],
            out_hbm.at[0, :, pl.ds(r, 1), :], sem).wait()


def kernel(relative_position_bias_table, relative_position_index):
    return _rpb_gather(relative_position_bias_table.reshape(-1),
                       relative_position_index)


# speedup vs baseline: 1.2049x; 1.0674x over previous
import functools

import jax
import jax.numpy as jnp
from jax import lax
from jax.experimental import pallas as pl
from jax.experimental.pallas import tpu as pltpu
from jax.experimental.pallas import tpu_sc as plsc

_WS = 16
_T = _WS * _WS
_H = 16
_NV = (2 * _WS - 1) ** 2
_NC = 2
_NS = 16
_NW = _NC * _NS
_R = _T // _NW
_L = 16
_G = _T // _L


def _make_mesh():
    return plsc.VectorSubcoreMesh(
        core_axis_name="c", subcore_axis_name="s",
        num_cores=_NC, num_subcores=_NS)


@functools.partial(
    pl.kernel,
    out_type=jax.ShapeDtypeStruct((1, _H, _T, _T), jnp.float32),
    mesh=_make_mesh(),
    scratch_types=[
        pltpu.VMEM((_R, _T), jnp.int32),
        pltpu.VMEM((_NV * _H,), jnp.float32),
        pltpu.VMEM((_H, _R, _T), jnp.float32),
        pltpu.SemaphoreType.DMA,
        pltpu.SemaphoreType.DMA,
    ],
    compiler_params=pltpu.CompilerParams(needs_layout_passes=False),
)
def _rpb_gather(tabt_hbm, idx_hbm, out_hbm, idx_v, tab_v, out_v, sem, sem_in):
    wid = lax.axis_index("s") * _NC + lax.axis_index("c")
    row0 = wid * _R
    tab_cp = pltpu.async_copy(tabt_hbm, tab_v, sem_in)
    idx_cp = pltpu.async_copy(idx_hbm.at[pl.ds(row0, _R), :], idx_v, sem_in)
    tab_cp.wait()
    idx_cp.wait()

    @pl.loop(0, _R)
    def _rows(r):
        @plsc.parallel_loop(0, _G, unroll=4)
        def _body(g):
            off = g * _L
            flat = idx_v[r, pl.ds(off, _L)] * _H
            for h in range(_H):
                out_v[h, r, pl.ds(off, _L)] = plsc.load_gather(
                    tab_v, [flat + h])
        pltpu.async_copy(
            out_v.at[:, pl.ds(r, 1), :],
            out_hbm.at[0, :, pl.ds(row0 + r, 1), :], sem)

    for r in range(_R):
        pltpu.make_async_copy(
            out_v.at[:, pl.ds(r, 1), :],
            out_hbm.at[0, :, pl.ds(r, 1), :], sem).wait()


def kernel(relative_position_bias_table, relative_position_index):
    return _rpb_gather(relative_position_bias_table.reshape(-1),
                       relative_position_index)


# speedup vs baseline: 1.2994x; 1.0785x over previous
import functools

import jax
import jax.numpy as jnp
from jax import lax
from jax.experimental import pallas as pl
from jax.experimental.pallas import tpu as pltpu
from jax.experimental.pallas import tpu_sc as plsc

_WS = 16
_T = _WS * _WS
_H = 16
_NV = (2 * _WS - 1) ** 2
_NC = 2
_NS = 16
_NW = _NC * _NS
_R = _T // _NW
_L = 16
_G = _T // _L


def _make_mesh():
    return plsc.VectorSubcoreMesh(
        core_axis_name="c", subcore_axis_name="s",
        num_cores=_NC, num_subcores=_NS)


@functools.partial(
    pl.kernel,
    out_type=jax.ShapeDtypeStruct((1, _H, _T, _T), jnp.float32),
    mesh=_make_mesh(),
    scratch_types=[
        pltpu.VMEM((_R, _T), jnp.int32),
        pltpu.VMEM((_NV * _H,), jnp.float32),
        pltpu.VMEM((_H, _R, _T), jnp.float32),
        pltpu.SemaphoreType.DMA,
        pltpu.SemaphoreType.DMA,
    ],
    compiler_params=pltpu.CompilerParams(needs_layout_passes=False),
)
def _rpb_gather(tabt_hbm, idx_hbm, out_hbm, idx_v, tab_v, out_v, sem, sem_in):
    wid = lax.axis_index("s") * _NC + lax.axis_index("c")
    row0 = wid * _R
    tab_cp = pltpu.async_copy(tabt_hbm, tab_v, sem_in)
    idx_cp = pltpu.async_copy(idx_hbm.at[pl.ds(row0, _R), :], idx_v, sem_in)
    tab_cp.wait()
    idx_cp.wait()

    @pl.loop(0, _R)
    def _rows(r):
        @plsc.parallel_loop(0, _G, unroll=2)
        def _body(g):
            off = g * _L
            flat = idx_v[r, pl.ds(off, _L)] * _H
            for h in range(_H):
                out_v[h, r, pl.ds(off, _L)] = plsc.load_gather(
                    tab_v, [flat + h])
        pltpu.async_copy(
            out_v.at[:, pl.ds(r, 1), :],
            out_hbm.at[0, :, pl.ds(row0 + r, 1), :], sem)

    for r in range(_R):
        pltpu.make_async_copy(
            out_v.at[:, pl.ds(r, 1), :],
            out_hbm.at[0, :, pl.ds(r, 1), :], sem).wait()


def kernel(relative_position_bias_table, relative_position_index):
    return _rpb_gather(relative_position_bias_table.reshape(-1),
                       relative_position_index)
